# baseline (speedup 1.0000x reference)
import functools

import jax
import jax.numpy as jnp
from jax import lax
from jax.experimental import pallas as pl
from jax.experimental.pallas import tpu as pltpu
from jax.experimental.pallas import tpu_sc as plsc

N = 10000
E = 320000
D = 128

NC = 2
NS = 16
NW = NC * NS
EPW = E // NW
CH = 125
NCHUNK = EPW // CH
RPT = N // NS
RCH = 5

DEG_ROWS = 640

_sc_mesh = functools.partial(
    plsc.VectorSubcoreMesh, core_axis_name="c", subcore_axis_name="s")
_sc_params = pltpu.CompilerParams(
    needs_layout_passes=False, use_tc_tiling_on_sc=False)


def _deg_body(dst_hbm, out_hbm, dst_v, acc_v):
  wid = lax.axis_index("s") * NC + lax.axis_index("c")
  pltpu.sync_copy(dst_hbm.at[wid], dst_v)

  def zero(i, _):
    acc_v[pl.ds(i * 16, 16)] = jnp.zeros((16,), jnp.float32)
    return 0
  lax.fori_loop(0, DEG_ROWS, zero, 0)

  ones = jnp.ones((16,), jnp.float32)

  def acc(i, _):
    d = dst_v[i, :]
    plsc.addupdate_scatter(acc_v, [d], ones)
    return 0
  lax.fori_loop(0, EPW // 16, acc, 0)

  pltpu.sync_copy(acc_v, out_hbm.at[wid])


def _deg_partials(dst):
  k = pl.kernel(
      _deg_body,
      out_type=jax.ShapeDtypeStruct((NW, DEG_ROWS * 16), jnp.float32),
      mesh=_sc_mesh(),
      scratch_types=[
          pltpu.VMEM((EPW // 16, 16), jnp.int32),
          pltpu.VMEM((DEG_ROWS * 16,), jnp.float32),
      ],
      compiler_params=_sc_params,
  )
  return k(dst)


def _prop_body(g_hbm, src_hbm, dst_hbm, out_hbm,
               sidx0, sidx1, didx0, didx1, r0_v, r1_v, acc_sh,
               g0, g1, ss0, ss1, sd0, sd1):
  c = lax.axis_index("c")
  s = lax.axis_index("s")
  wid = s * NC + c

  hs0 = pltpu.async_copy(src_hbm.at[wid, 0], sidx0, ss0)
  pltpu.async_copy(src_hbm.at[wid, 1], sidx1, ss1)
  pltpu.async_copy(dst_hbm.at[wid, 0], didx0, sd0)
  pltpu.async_copy(dst_hbm.at[wid, 1], didx1, sd1)

  def zero(i, _):
    for b in range(D // 16):
      r0_v[i, pl.ds(b * 16, 16)] = jnp.zeros((16,), jnp.float32)
    return 0
  lax.fori_loop(0, CH, zero, 0)
  for k in range(RCH):
    pltpu.sync_copy(r0_v, acc_sh.at[pl.ds(s * RPT + k * CH, CH)])
  plsc.subcore_barrier()

  hs0.wait()
  pltpu.async_copy(g_hbm.at[sidx0], r0_v, g0).wait()

  def step(t, _):
    j = 2 * t
    pltpu.make_async_copy(src_hbm.at[wid, 0], sidx1, ss1).wait()
    hb = pltpu.async_copy(g_hbm.at[sidx1], r1_v, g1)
    hsc = pltpu.async_copy(src_hbm.at[wid, j + 2], sidx0, ss0)
    pltpu.make_async_copy(dst_hbm.at[wid, 0], didx0, sd0).wait()
    pltpu.sync_copy(r0_v, acc_sh.at[didx0], add=True)
    pltpu.async_copy(dst_hbm.at[wid, j + 2], didx0, sd0)
    hb.wait()
    hsc.wait()
    hc = pltpu.async_copy(g_hbm.at[sidx0], r0_v, g0)
    pltpu.async_copy(src_hbm.at[wid, j + 3], sidx1, ss1)
    pltpu.make_async_copy(dst_hbm.at[wid, 0], didx1, sd1).wait()
    pltpu.sync_copy(r1_v, acc_sh.at[didx1], add=True)
    pltpu.async_copy(dst_hbm.at[wid, j + 3], didx1, sd1)
    hc.wait()
    return 0
  lax.fori_loop(0, NCHUNK // 2, step, 0)

  pltpu.make_async_copy(dst_hbm.at[wid, 0], didx0, sd0).wait()
  pltpu.make_async_copy(src_hbm.at[wid, 0], sidx1, ss1).wait()
  pltpu.make_async_copy(dst_hbm.at[wid, 0], didx1, sd1).wait()
  plsc.subcore_barrier()

  for k in range(RCH):
    r0 = s * RPT + k * CH
    pltpu.sync_copy(acc_sh.at[pl.ds(r0, CH)], r0_v)
    pltpu.sync_copy(r0_v, out_hbm.at[c].at[pl.ds(r0, CH)])


def _propagate(g, src, dst):
  k = pl.kernel(
      _prop_body,
      out_type=jax.ShapeDtypeStruct((NC, N, D), jnp.float32),
      mesh=_sc_mesh(),
      scratch_types=[
          pltpu.VMEM((CH,), jnp.int32),
          pltpu.VMEM((CH,), jnp.int32),
          pltpu.VMEM((CH,), jnp.int32),
          pltpu.VMEM((CH,), jnp.int32),
          pltpu.VMEM((CH, D), jnp.float32),
          pltpu.VMEM((CH, D), jnp.float32),
          pltpu.VMEM_SHARED((N, D), jnp.float32),
      ] + [pltpu.SemaphoreType.DMA] * 6,
      compiler_params=_sc_params,
  )
  return k(g, src, dst)


def _tc1_body(x_ref, w_ref, dinv_ref, o_ref):
  h = jnp.dot(x_ref[...], w_ref[...], preferred_element_type=jnp.float32)
  o_ref[...] = h * dinv_ref[...]


def _tc2_body(p_ref, g_ref, dinv_ref, b_ref, w_ref, o_ref):
  agg = (p_ref[0] + p_ref[1] + g_ref[...]) * dinv_ref[...] + b_ref[...]
  h1 = jnp.maximum(agg, 0.0)
  h2 = jnp.dot(h1, w_ref[...], preferred_element_type=jnp.float32)
  o_ref[...] = h2 * dinv_ref[...]


def _tc3_body(p_ref, g_ref, dinv_ref, b_ref, o_ref):
  o_ref[...] = (p_ref[0] + p_ref[1] + g_ref[...]) * dinv_ref[...] + b_ref[...]


def _tc_call(body, *args):
  return pl.pallas_call(
      body, out_shape=jax.ShapeDtypeStruct((N, D), jnp.float32))(*args)


@jax.jit
def kernel(x, edge_index, W1, b1, W2, b2):
  pad = jnp.zeros((NW, 2, CH), jnp.int32)
  src = jnp.concatenate([edge_index[0].reshape(NW, NCHUNK, CH), pad], axis=1)
  dst = jnp.concatenate([edge_index[1].reshape(NW, NCHUNK, CH), pad], axis=1)
  dst16 = edge_index[1].reshape(NW, EPW // 16, 16)

  degp = _deg_partials(dst16)
  deg = 1.0 + jnp.sum(degp, axis=0)[:N]
  dinv = lax.rsqrt(deg).reshape(N, 1)

  g1 = _tc_call(_tc1_body, x, W1, dinv)
  p1 = _propagate(g1, src, dst)
  g2 = _tc_call(_tc2_body, p1, g1, dinv, b1.reshape(1, D), W2)
  p2 = _propagate(g2, src, dst)
  out = _tc_call(_tc3_body, p2, g2, dinv, b2.reshape(1, D))
  return out


# speedup vs baseline: 1.5234x; 1.5234x over previous
import functools

import jax
import jax.numpy as jnp
from jax import lax
from jax.experimental import pallas as pl
from jax.experimental.pallas import tpu as pltpu
from jax.experimental.pallas import tpu_sc as plsc

N = 10000
E = 320000
D = 128

NC = 2
NS = 16
NW = NC * NS
EPW = E // NW
CH = 50
NCHUNK = EPW // CH
NBUF = 4
RPT = N // NS
RCH = 5
ZCH = 125

DEG_ROWS = 640

_sc_mesh = functools.partial(
    plsc.VectorSubcoreMesh, core_axis_name="c", subcore_axis_name="s")
_sc_params = pltpu.CompilerParams(
    needs_layout_passes=False, use_tc_tiling_on_sc=False)


def _deg_body(dst_hbm, out_hbm, dst_v, acc_v):
  wid = lax.axis_index("s") * NC + lax.axis_index("c")
  pltpu.sync_copy(dst_hbm.at[wid], dst_v)

  def zero(i, _):
    acc_v[pl.ds(i * 16, 16)] = jnp.zeros((16,), jnp.float32)
    return 0
  lax.fori_loop(0, DEG_ROWS, zero, 0)

  ones = jnp.ones((16,), jnp.float32)

  def acc(i, _):
    d = dst_v[i, :]
    plsc.addupdate_scatter(acc_v, [d], ones)
    return 0
  lax.fori_loop(0, EPW // 16, acc, 0)

  pltpu.sync_copy(acc_v, out_hbm.at[wid])


def _deg_partials(dst):
  k = pl.kernel(
      _deg_body,
      out_type=jax.ShapeDtypeStruct((NW, DEG_ROWS * 16), jnp.float32),
      mesh=_sc_mesh(),
      scratch_types=[
          pltpu.VMEM((EPW // 16, 16), jnp.int32),
          pltpu.VMEM((DEG_ROWS * 16,), jnp.float32),
      ],
      compiler_params=_sc_params,
  )
  return k(dst)


def _prop_body(g_hbm, src_hbm, dst_hbm, out_hbm, src_v, dst_v,
               r0_v, r1_v, r2_v, r3_v, acc_sh, g0, g1, g2, g3):
  rows = (r0_v, r1_v, r2_v, r3_v)
  gsem = (g0, g1, g2, g3)
  c = lax.axis_index("c")
  s = lax.axis_index("s")
  wid = s * NC + c
  pltpu.sync_copy(src_hbm.at[wid], src_v)
  pltpu.sync_copy(dst_hbm.at[wid], dst_v)

  def zero(i, _):
    for b in range(D // 16):
      r0_v[i, pl.ds(b * 16, 16)] = jnp.zeros((16,), jnp.float32)
    return 0
  lax.fori_loop(0, CH, zero, 0)
  for k in range(12):
    pltpu.sync_copy(r0_v, acc_sh.at[pl.ds(s * RPT + k * CH, CH)])
  pltpu.sync_copy(r0_v.at[pl.ds(0, RPT - 12 * CH)],
                  acc_sh.at[pl.ds(s * RPT + 12 * CH, RPT - 12 * CH)])
  plsc.subcore_barrier()

  def step(t, _):
    j = t * NBUF
    hs = [pltpu.async_copy(g_hbm.at[src_v.at[j + b]], rows[b], gsem[b])
          for b in range(NBUF)]
    for b in range(NBUF):
      hs[b].wait()
      pltpu.sync_copy(rows[b], acc_sh.at[dst_v.at[j + b]], add=True)
    return 0
  lax.fori_loop(0, NCHUNK // NBUF, step, 0)
  plsc.subcore_barrier()

  for k in range(12):
    r0 = s * RPT + k * CH
    pltpu.sync_copy(acc_sh.at[pl.ds(r0, CH)], r0_v)
    pltpu.sync_copy(r0_v, out_hbm.at[c].at[pl.ds(r0, CH)])
  rr = RPT - 12 * CH
  pltpu.sync_copy(acc_sh.at[pl.ds(s * RPT + 12 * CH, rr)],
                  r0_v.at[pl.ds(0, rr)])
  pltpu.sync_copy(r0_v.at[pl.ds(0, rr)],
                  out_hbm.at[c].at[pl.ds(s * RPT + 12 * CH, rr)])


def _propagate(g, src, dst):
  k = pl.kernel(
      _prop_body,
      out_type=jax.ShapeDtypeStruct((NC, N, D), jnp.float32),
      mesh=_sc_mesh(),
      scratch_types=[
          pltpu.VMEM((NCHUNK, CH), jnp.int32),
          pltpu.VMEM((NCHUNK, CH), jnp.int32),
          pltpu.VMEM((CH, D), jnp.float32),
          pltpu.VMEM((CH, D), jnp.float32),
          pltpu.VMEM((CH, D), jnp.float32),
          pltpu.VMEM((CH, D), jnp.float32),
          pltpu.VMEM_SHARED((N, D), jnp.float32),
      ] + [pltpu.SemaphoreType.DMA] * NBUF,
      compiler_params=_sc_params,
  )
  return k(g, src, dst)


def _tc1_body(x_ref, w_ref, dinv_ref, o_ref):
  h = jnp.dot(x_ref[...], w_ref[...], preferred_element_type=jnp.float32)
  o_ref[...] = h * dinv_ref[...]


def _tc2_body(p_ref, g_ref, dinv_ref, b_ref, w_ref, o_ref):
  agg = (p_ref[0] + p_ref[1] + g_ref[...]) * dinv_ref[...] + b_ref[...]
  h1 = jnp.maximum(agg, 0.0)
  h2 = jnp.dot(h1, w_ref[...], preferred_element_type=jnp.float32)
  o_ref[...] = h2 * dinv_ref[...]


def _tc3_body(p_ref, g_ref, dinv_ref, b_ref, o_ref):
  o_ref[...] = (p_ref[0] + p_ref[1] + g_ref[...]) * dinv_ref[...] + b_ref[...]


def _tc_call(body, *args):
  return pl.pallas_call(
      body, out_shape=jax.ShapeDtypeStruct((N, D), jnp.float32))(*args)


@jax.jit
def kernel(x, edge_index, W1, b1, W2, b2):
  src = edge_index[0].reshape(NW, NCHUNK, CH)
  dst = edge_index[1].reshape(NW, NCHUNK, CH)
  dst16 = edge_index[1].reshape(NW, EPW // 16, 16)

  degp = _deg_partials(dst16)
  deg = 1.0 + jnp.sum(degp, axis=0)[:N]
  dinv = lax.rsqrt(deg).reshape(N, 1)

  g1 = _tc_call(_tc1_body, x, W1, dinv)
  p1 = _propagate(g1, src, dst)
  g2 = _tc_call(_tc2_body, p1, g1, dinv, b1.reshape(1, D), W2)
  p2 = _propagate(g2, src, dst)
  out = _tc_call(_tc3_body, p2, g2, dinv, b2.reshape(1, D))
  return out


# speedup vs baseline: 1.6282x; 1.0688x over previous
import functools

import jax
import jax.numpy as jnp
from jax import lax
from jax.experimental import pallas as pl
from jax.experimental.pallas import tpu as pltpu
from jax.experimental.pallas import tpu_sc as plsc

N = 10000
E = 320000
D = 128

NC = 2
NS = 16
NW = NC * NS
EPW = E // NW
CH = 100
NCHUNK = EPW // CH
NBUF = 2
RPT = N // NS
RCH = 5
ZCH = 125

DEG_ROWS = 640

_sc_mesh = functools.partial(
    plsc.VectorSubcoreMesh, core_axis_name="c", subcore_axis_name="s")
_sc_params = pltpu.CompilerParams(
    needs_layout_passes=False, use_tc_tiling_on_sc=False)


def _deg_body(dst_hbm, out_hbm, dst_v, acc_v):
  wid = lax.axis_index("s") * NC + lax.axis_index("c")
  pltpu.sync_copy(dst_hbm.at[wid], dst_v)

  def zero(i, _):
    acc_v[pl.ds(i * 16, 16)] = jnp.zeros((16,), jnp.float32)
    return 0
  lax.fori_loop(0, DEG_ROWS, zero, 0)

  ones = jnp.ones((16,), jnp.float32)

  def acc(i, _):
    d = dst_v[i, :]
    plsc.addupdate_scatter(acc_v, [d], ones)
    return 0
  lax.fori_loop(0, EPW // 16, acc, 0)

  pltpu.sync_copy(acc_v, out_hbm.at[wid])


def _deg_partials(dst):
  k = pl.kernel(
      _deg_body,
      out_type=jax.ShapeDtypeStruct((NW, DEG_ROWS * 16), jnp.float32),
      mesh=_sc_mesh(),
      scratch_types=[
          pltpu.VMEM((EPW // 16, 16), jnp.int32),
          pltpu.VMEM((DEG_ROWS * 16,), jnp.float32),
      ],
      compiler_params=_sc_params,
  )
  return k(dst)


def _prop_body(g_hbm, src_hbm, dst_hbm, out_hbm, src_v, dst_v,
               r0_v, r1_v, acc_sh, g0, g1):
  rows = (r0_v, r1_v)
  gsem = (g0, g1)
  c = lax.axis_index("c")
  s = lax.axis_index("s")
  wid = s * NC + c
  pltpu.sync_copy(src_hbm.at[wid], src_v)
  pltpu.sync_copy(dst_hbm.at[wid], dst_v)

  def zero(i, _):
    for b in range(D // 16):
      r0_v[i, pl.ds(b * 16, 16)] = jnp.zeros((16,), jnp.float32)
    return 0
  lax.fori_loop(0, CH, zero, 0)
  for k in range(6):
    pltpu.sync_copy(r0_v, acc_sh.at[pl.ds(s * RPT + k * CH, CH)])
  pltpu.sync_copy(r0_v.at[pl.ds(0, RPT - 6 * CH)],
                  acc_sh.at[pl.ds(s * RPT + 6 * CH, RPT - 6 * CH)])
  plsc.subcore_barrier()

  def step(t, _):
    j = t * NBUF
    hs = [pltpu.async_copy(g_hbm.at[src_v.at[j + b]], rows[b], gsem[b])
          for b in range(NBUF)]
    for b in range(NBUF):
      hs[b].wait()
      pltpu.sync_copy(rows[b], acc_sh.at[dst_v.at[j + b]], add=True)
    return 0
  lax.fori_loop(0, NCHUNK // NBUF, step, 0)
  plsc.subcore_barrier()

  for k in range(6):
    r0 = s * RPT + k * CH
    pltpu.sync_copy(acc_sh.at[pl.ds(r0, CH)], r0_v)
    pltpu.sync_copy(r0_v, out_hbm.at[c].at[pl.ds(r0, CH)])
  rr = RPT - 6 * CH
  pltpu.sync_copy(acc_sh.at[pl.ds(s * RPT + 6 * CH, rr)],
                  r0_v.at[pl.ds(0, rr)])
  pltpu.sync_copy(r0_v.at[pl.ds(0, rr)],
                  out_hbm.at[c].at[pl.ds(s * RPT + 6 * CH, rr)])


def _propagate(g, src, dst):
  k = pl.kernel(
      _prop_body,
      out_type=jax.ShapeDtypeStruct((NC, N, D), jnp.float32),
      mesh=_sc_mesh(),
      scratch_types=[
          pltpu.VMEM((NCHUNK, CH), jnp.int32),
          pltpu.VMEM((NCHUNK, CH), jnp.int32),
          pltpu.VMEM((CH, D), jnp.float32),
          pltpu.VMEM((CH, D), jnp.float32),
          pltpu.VMEM_SHARED((N, D), jnp.float32),
      ] + [pltpu.SemaphoreType.DMA] * NBUF,
      compiler_params=_sc_params,
  )
  return k(g, src, dst)


def _tc1_body(x_ref, w_ref, dinv_ref, o_ref):
  h = jnp.dot(x_ref[...], w_ref[...], preferred_element_type=jnp.float32)
  o_ref[...] = h * dinv_ref[...]


def _tc2_body(p_ref, g_ref, dinv_ref, b_ref, w_ref, o_ref):
  agg = (p_ref[0] + p_ref[1] + g_ref[...]) * dinv_ref[...] + b_ref[...]
  h1 = jnp.maximum(agg, 0.0)
  h2 = jnp.dot(h1, w_ref[...], preferred_element_type=jnp.float32)
  o_ref[...] = h2 * dinv_ref[...]


def _tc3_body(p_ref, g_ref, dinv_ref, b_ref, o_ref):
  o_ref[...] = (p_ref[0] + p_ref[1] + g_ref[...]) * dinv_ref[...] + b_ref[...]


def _tc_call(body, *args):
  return pl.pallas_call(
      body, out_shape=jax.ShapeDtypeStruct((N, D), jnp.float32))(*args)


@jax.jit
def kernel(x, edge_index, W1, b1, W2, b2):
  src = edge_index[0].reshape(NW, NCHUNK, CH)
  dst = edge_index[1].reshape(NW, NCHUNK, CH)
  dst16 = edge_index[1].reshape(NW, EPW // 16, 16)

  degp = _deg_partials(dst16)
  deg = 1.0 + jnp.sum(degp, axis=0)[:N]
  dinv = lax.rsqrt(deg).reshape(N, 1)

  g1 = _tc_call(_tc1_body, x, W1, dinv)
  p1 = _propagate(g1, src, dst)
  g2 = _tc_call(_tc2_body, p1, g1, dinv, b1.reshape(1, D), W2)
  p2 = _propagate(g2, src, dst)
  out = _tc_call(_tc3_body, p2, g2, dinv, b2.reshape(1, D))
  return out


# speedup vs baseline: 1.6537x; 1.0156x over previous
import functools

import jax
import jax.numpy as jnp
from jax import lax
from jax.experimental import pallas as pl
from jax.experimental.pallas import tpu as pltpu
from jax.experimental.pallas import tpu_sc as plsc

N = 10000
E = 320000
D = 128

NC = 2
NS = 16
NW = NC * NS
EPW = E // NW
CH = 100
NCHUNK = EPW // CH
NBUF = 2
RPT = N // NS
RCH = 5
ZCH = 125

DEG_ROWS = 640

_sc_mesh = functools.partial(
    plsc.VectorSubcoreMesh, core_axis_name="c", subcore_axis_name="s")
_sc_params = pltpu.CompilerParams(
    needs_layout_passes=False, use_tc_tiling_on_sc=False)


def _deg_body(dst_hbm, out_hbm, dst_v, acc_v):
  wid = lax.axis_index("s") * NC + lax.axis_index("c")
  pltpu.sync_copy(dst_hbm.at[wid], dst_v)

  def zero(i, _):
    acc_v[pl.ds(i * 16, 16)] = jnp.zeros((16,), jnp.float32)
    return 0
  lax.fori_loop(0, DEG_ROWS, zero, 0)

  ones = jnp.ones((16,), jnp.float32)

  def acc(i, _):
    d = dst_v[i, :]
    plsc.addupdate_scatter(acc_v, [d], ones)
    return 0
  lax.fori_loop(0, EPW // 16, acc, 0)

  pltpu.sync_copy(acc_v, out_hbm.at[wid])


def _deg_partials(dst):
  k = pl.kernel(
      _deg_body,
      out_type=jax.ShapeDtypeStruct((NW, DEG_ROWS * 16), jnp.float32),
      mesh=_sc_mesh(),
      scratch_types=[
          pltpu.VMEM((EPW // 16, 16), jnp.int32),
          pltpu.VMEM((DEG_ROWS * 16,), jnp.float32),
      ],
      compiler_params=_sc_params,
  )
  return k(dst)


def _prop_body(g_hbm, src_hbm, dst_hbm, out_hbm, src_v, dst_v,
               r0_v, r1_v, acc_sh, g0, g1, s0, s1):
  rows = (r0_v, r1_v)
  gsem = (g0, g1)
  ssem = (s0, s1)
  c = lax.axis_index("c")
  s = lax.axis_index("s")
  wid = s * NC + c
  pltpu.sync_copy(src_hbm.at[wid], src_v)
  pltpu.sync_copy(dst_hbm.at[wid], dst_v)

  def zero(i, _):
    for b in range(D // 16):
      r0_v[i, pl.ds(b * 16, 16)] = jnp.zeros((16,), jnp.float32)
    return 0
  lax.fori_loop(0, CH, zero, 0)
  for k in range(6):
    pltpu.sync_copy(r0_v, acc_sh.at[pl.ds(s * RPT + k * CH, CH)])
  pltpu.sync_copy(r0_v.at[pl.ds(0, RPT - 6 * CH)],
                  acc_sh.at[pl.ds(s * RPT + 6 * CH, RPT - 6 * CH)])
  plsc.subcore_barrier()

  def step(t, _):
    j = t * NBUF
    hs = [pltpu.async_copy(g_hbm.at[src_v.at[j + b]], rows[b], gsem[b])
          for b in range(NBUF)]
    sc = []
    for b in range(NBUF):
      hs[b].wait()
      sc.append(pltpu.async_copy(rows[b], acc_sh.at[dst_v.at[j + b]],
                                 ssem[b], add=True))
    for b in range(NBUF):
      sc[b].wait()
    return 0
  lax.fori_loop(0, NCHUNK // NBUF, step, 0)
  plsc.subcore_barrier()

  for k in range(6):
    r0 = s * RPT + k * CH
    pltpu.sync_copy(acc_sh.at[pl.ds(r0, CH)], r0_v)
    pltpu.sync_copy(r0_v, out_hbm.at[c].at[pl.ds(r0, CH)])
  rr = RPT - 6 * CH
  pltpu.sync_copy(acc_sh.at[pl.ds(s * RPT + 6 * CH, rr)],
                  r0_v.at[pl.ds(0, rr)])
  pltpu.sync_copy(r0_v.at[pl.ds(0, rr)],
                  out_hbm.at[c].at[pl.ds(s * RPT + 6 * CH, rr)])


def _propagate(g, src, dst):
  k = pl.kernel(
      _prop_body,
      out_type=jax.ShapeDtypeStruct((NC, N, D), jnp.float32),
      mesh=_sc_mesh(),
      scratch_types=[
          pltpu.VMEM((NCHUNK, CH), jnp.int32),
          pltpu.VMEM((NCHUNK, CH), jnp.int32),
          pltpu.VMEM((CH, D), jnp.float32),
          pltpu.VMEM((CH, D), jnp.float32),
          pltpu.VMEM_SHARED((N, D), jnp.float32),
      ] + [pltpu.SemaphoreType.DMA] * (2 * NBUF),
      compiler_params=_sc_params,
  )
  return k(g, src, dst)


def _tc1_body(x_ref, w_ref, dinv_ref, o_ref):
  h = jnp.dot(x_ref[...], w_ref[...], preferred_element_type=jnp.float32)
  o_ref[...] = h * dinv_ref[...]


def _tc2_body(p_ref, g_ref, dinv_ref, b_ref, w_ref, o_ref):
  agg = (p_ref[0] + p_ref[1] + g_ref[...]) * dinv_ref[...] + b_ref[...]
  h1 = jnp.maximum(agg, 0.0)
  h2 = jnp.dot(h1, w_ref[...], preferred_element_type=jnp.float32)
  o_ref[...] = h2 * dinv_ref[...]


def _tc3_body(p_ref, g_ref, dinv_ref, b_ref, o_ref):
  o_ref[...] = (p_ref[0] + p_ref[1] + g_ref[...]) * dinv_ref[...] + b_ref[...]


def _tc_call(body, *args):
  return pl.pallas_call(
      body, out_shape=jax.ShapeDtypeStruct((N, D), jnp.float32))(*args)


@jax.jit
def kernel(x, edge_index, W1, b1, W2, b2):
  src = edge_index[0].reshape(NW, NCHUNK, CH)
  dst = edge_index[1].reshape(NW, NCHUNK, CH)
  dst16 = edge_index[1].reshape(NW, EPW // 16, 16)

  degp = _deg_partials(dst16)
  deg = 1.0 + jnp.sum(degp, axis=0)[:N]
  dinv = lax.rsqrt(deg).reshape(N, 1)

  g1 = _tc_call(_tc1_body, x, W1, dinv)
  p1 = _propagate(g1, src, dst)
  g2 = _tc_call(_tc2_body, p1, g1, dinv, b1.reshape(1, D), W2)
  p2 = _propagate(g2, src, dst)
  out = _tc_call(_tc3_body, p2, g2, dinv, b2.reshape(1, D))
  return out


# speedup vs baseline: 1.6595x; 1.0035x over previous
import functools

import jax
import jax.numpy as jnp
from jax import lax
from jax.experimental import pallas as pl
from jax.experimental.pallas import tpu as pltpu
from jax.experimental.pallas import tpu_sc as plsc

N = 10000
E = 320000
D = 128

NC = 2
NS = 16
NW = NC * NS
EPW = E // NW
CH = 100
NCHUNK = EPW // CH
NBUF = 2
RPT = N // NS
RCH = 5
ZCH = 125

DEG_ROWS = 640

_sc_mesh = functools.partial(
    plsc.VectorSubcoreMesh, core_axis_name="c", subcore_axis_name="s")
_sc_params = pltpu.CompilerParams(
    needs_layout_passes=False, use_tc_tiling_on_sc=False)


def _deg_body(dst_hbm, out_hbm, dst_v, acc_v):
  wid = lax.axis_index("s") * NC + lax.axis_index("c")
  pltpu.sync_copy(dst_hbm.at[wid], dst_v)

  def zero(i, _):
    acc_v[pl.ds(i * 16, 16)] = jnp.zeros((16,), jnp.float32)
    return 0
  lax.fori_loop(0, DEG_ROWS, zero, 0)

  ones = jnp.ones((16,), jnp.float32)

  def acc(i, _):
    d = dst_v[i, :]
    plsc.addupdate_scatter(acc_v, [d], ones)
    return 0
  lax.fori_loop(0, EPW // 16, acc, 0)

  pltpu.sync_copy(acc_v, out_hbm.at[wid])


def _deg_partials(dst):
  k = pl.kernel(
      _deg_body,
      out_type=jax.ShapeDtypeStruct((NW, DEG_ROWS * 16), jnp.float32),
      mesh=_sc_mesh(),
      scratch_types=[
          pltpu.VMEM((EPW // 16, 16), jnp.int32),
          pltpu.VMEM((DEG_ROWS * 16,), jnp.float32),
      ],
      compiler_params=_sc_params,
  )
  return k(dst)


def _prop_body(g_hbm, src_hbm, dst_hbm, out_hbm, src_v, dst_v,
               r0_v, r1_v, acc_sh, g0, g1, s0, s1):
  rows = (r0_v, r1_v)
  gsem = (g0, g1)
  ssem = (s0, s1)
  c = lax.axis_index("c")
  s = lax.axis_index("s")
  wid = s * NC + c
  pltpu.sync_copy(src_hbm.at[wid], src_v)
  pltpu.sync_copy(dst_hbm.at[wid], dst_v)

  def zero(i, _):
    for b in range(D // 16):
      r0_v[i, pl.ds(b * 16, 16)] = jnp.zeros((16,), jnp.float32)
    return 0
  lax.fori_loop(0, CH, zero, 0)
  for k in range(6):
    pltpu.sync_copy(r0_v, acc_sh.at[pl.ds(s * RPT + k * CH, CH)])
  pltpu.sync_copy(r0_v.at[pl.ds(0, RPT - 6 * CH)],
                  acc_sh.at[pl.ds(s * RPT + 6 * CH, RPT - 6 * CH)])
  plsc.subcore_barrier()

  def step(t, _):
    j = t * NBUF
    hs = [pltpu.async_copy(g_hbm.at[src_v.at[j + b]], rows[b], gsem[b])
          for b in range(NBUF)]
    sc = []
    for b in range(NBUF):
      hs[b].wait()
      sc.append(pltpu.async_copy(rows[b], acc_sh.at[dst_v.at[j + b]],
                                 ssem[b], add=True))
    for b in range(NBUF):
      sc[b].wait()
    return 0
  lax.fori_loop(0, NCHUNK // NBUF, step, 0)
  plsc.subcore_barrier()

  pltpu.sync_copy(acc_sh.at[pl.ds(s * RPT, RPT)],
                  out_hbm.at[c].at[pl.ds(s * RPT, RPT)])


def _propagate(g, src, dst):
  k = pl.kernel(
      _prop_body,
      out_type=jax.ShapeDtypeStruct((NC, N, D), jnp.float32),
      mesh=_sc_mesh(),
      scratch_types=[
          pltpu.VMEM((NCHUNK, CH), jnp.int32),
          pltpu.VMEM((NCHUNK, CH), jnp.int32),
          pltpu.VMEM((CH, D), jnp.float32),
          pltpu.VMEM((CH, D), jnp.float32),
          pltpu.VMEM_SHARED((N, D), jnp.float32),
      ] + [pltpu.SemaphoreType.DMA] * (2 * NBUF),
      compiler_params=_sc_params,
  )
  return k(g, src, dst)


def _tc1_body(x_ref, w_ref, dinv_ref, o_ref):
  h = jnp.dot(x_ref[...], w_ref[...], preferred_element_type=jnp.float32)
  o_ref[...] = h * dinv_ref[...]


def _tc2_body(p_ref, g_ref, dinv_ref, b_ref, w_ref, o_ref):
  agg = (p_ref[0] + p_ref[1] + g_ref[...]) * dinv_ref[...] + b_ref[...]
  h1 = jnp.maximum(agg, 0.0)
  h2 = jnp.dot(h1, w_ref[...], preferred_element_type=jnp.float32)
  o_ref[...] = h2 * dinv_ref[...]


def _tc3_body(p_ref, g_ref, dinv_ref, b_ref, o_ref):
  o_ref[...] = (p_ref[0] + p_ref[1] + g_ref[...]) * dinv_ref[...] + b_ref[...]


def _tc_call(body, *args):
  return pl.pallas_call(
      body, out_shape=jax.ShapeDtypeStruct((N, D), jnp.float32))(*args)


@jax.jit
def kernel(x, edge_index, W1, b1, W2, b2):
  src = edge_index[0].reshape(NW, NCHUNK, CH)
  dst = edge_index[1].reshape(NW, NCHUNK, CH)
  dst16 = edge_index[1].reshape(NW, EPW // 16, 16)

  degp = _deg_partials(dst16)
  deg = 1.0 + jnp.sum(degp, axis=0)[:N]
  dinv = lax.rsqrt(deg).reshape(N, 1)

  g1 = _tc_call(_tc1_body, x, W1, dinv)
  p1 = _propagate(g1, src, dst)
  g2 = _tc_call(_tc2_body, p1, g1, dinv, b1.reshape(1, D), W2)
  p2 = _propagate(g2, src, dst)
  out = _tc_call(_tc3_body, p2, g2, dinv, b2.reshape(1, D))
  return out
